# baseline (speedup 1.0000x reference)
import functools

import jax
import jax.numpy as jnp
from jax import lax
from jax.experimental import pallas as pl
from jax.experimental.pallas import tpu as pltpu
from jax.experimental.pallas import tpu_sc as plsc

_NUM_CODE = 8192
_DIM = 32
_TM = 128


_HALF = _NUM_CODE // 2


def _half_min_arg(sz, se_h, z16, emb2_h, base):
    dims = (((1,), (1,)), ((), ()))
    mm2 = lax.dot_general(z16, emb2_h, dims,
                          preferred_element_type=jnp.float32)
    d = (sz + se_h) - mm2
    m = jnp.min(d, axis=1, keepdims=True)
    iota = lax.broadcasted_iota(jnp.int32, d.shape, 1) + base
    arg = jnp.min(jnp.where(d == m, iota, jnp.int32(2 ** 30)), axis=1,
                  keepdims=True)
    return m, arg


def _dist_argmin_body(z16_ref, emb2_ref, sz_ref, se_ref, idx_ref, loss_ref):
    i = pl.program_id(0)
    z16 = z16_ref[...]
    emb2 = emb2_ref[...]
    sz = sz_ref[...]
    se = se_ref[...]
    m1, a1 = _half_min_arg(sz, se[:, :_HALF], z16, emb2[:_HALF], 0)
    m2, a2 = _half_min_arg(sz, se[:, _HALF:], z16, emb2[_HALF:], _HALF)
    m1b = m1.astype(jnp.bfloat16).astype(jnp.float32)
    take2 = m2 < m1b
    idx_ref[...] = jnp.where(take2, a2, a1)[:, 0]
    chosen = jnp.where(take2, m2, m1)
    prev = jnp.where(i == 0, 0.0, loss_ref[0, 0])
    loss_ref[0, 0] = prev + jnp.sum(chosen)


def _tc_argmin(z_flat, embedding):
    n_tok = z_flat.shape[0]
    sz = jnp.sum(z_flat ** 2, axis=1, keepdims=True)
    se = jnp.sum(embedding ** 2, axis=1)[None, :]
    z16 = z_flat.astype(jnp.bfloat16)
    emb2 = embedding + embedding
    return pl.pallas_call(
        _dist_argmin_body,
        grid=(n_tok // _TM,),
        in_specs=[pl.BlockSpec((_TM, _DIM), lambda i: (i, 0)),
                  pl.BlockSpec((_NUM_CODE, _DIM), lambda i: (0, 0)),
                  pl.BlockSpec((_TM, 1), lambda i: (i, 0)),
                  pl.BlockSpec((1, _NUM_CODE), lambda i: (0, 0))],
        out_specs=[pl.BlockSpec((_TM,), lambda i: (i,)),
                   pl.BlockSpec((1, 1), lambda i: (0, 0),
                                memory_space=pltpu.SMEM)],
        out_shape=[jax.ShapeDtypeStruct((n_tok,), jnp.int32),
                   jax.ShapeDtypeStruct((1, 1), jnp.float32)],
        compiler_params=pltpu.CompilerParams(
            dimension_semantics=("arbitrary",)),
    )(z16, emb2, sz, se)


_GW = 128


def _sc_gather(emb_pad, idx):
    n_tok = idx.shape[0]
    info = plsc.get_sparse_core_info()
    nc, ns = info.num_cores, info.num_subcores
    nw = nc * ns
    bpw = n_tok // nw
    ch = 128
    mesh = plsc.VectorSubcoreMesh(core_axis_name="c", subcore_axis_name="s")

    @functools.partial(
        pl.kernel, mesh=mesh,
        out_type=jax.ShapeDtypeStruct((n_tok, _GW), jnp.float32),
        scratch_types=[pltpu.VMEM((bpw,), jnp.int32),
                       pltpu.VMEM((bpw, _GW), jnp.float32),
                       pltpu.SemaphoreType.DMA],
    )
    def k(emb_hbm, idx_hbm, out_hbm, idx_v, rows_v, sem):
        wid = lax.axis_index("s") * nc + lax.axis_index("c")
        base = wid * bpw
        pltpu.sync_copy(idx_hbm.at[pl.ds(base, bpw)], idx_v)
        for j in range(bpw // ch):
            pltpu.async_copy(emb_hbm.at[idx_v.at[pl.ds(j * ch, ch)]],
                             rows_v.at[pl.ds(j * ch, ch)], sem).wait()
        pltpu.sync_copy(rows_v, out_hbm.at[pl.ds(base, bpw)])

    return k(emb_pad, idx)


def kernel(z, embedding):
    b, n, d = z.shape
    z_flat = z.reshape(-1, d)
    idx, loss_sum = _tc_argmin(z_flat, embedding)
    emb_pad = jnp.pad(embedding, ((0, 0), (0, _GW - d)))
    z_q = _sc_gather(emb_pad, idx)[:, :d].reshape(b, n, d)
    loss = loss_sum[0, 0] * (2.0 / z_flat.size)
    z_q_st = z + (z_q - z)
    return (z_q_st, idx.reshape(b, n), loss)


# speedup vs baseline: 1.0177x; 1.0177x over previous
import functools

import jax
import jax.numpy as jnp
from jax import lax
from jax.experimental import pallas as pl
from jax.experimental.pallas import tpu as pltpu
from jax.experimental.pallas import tpu_sc as plsc

_NUM_CODE = 8192
_DIM = 32
_TM = 128


_HALF = _NUM_CODE // 2


def _half_min_arg(sz, se_h, z16, emb2_h, base):
    dims = (((1,), (1,)), ((), ()))
    mm2 = lax.dot_general(z16, emb2_h, dims,
                          preferred_element_type=jnp.float32)
    d = (sz + se_h) - mm2
    m = jnp.min(d, axis=1, keepdims=True)
    iota = lax.broadcasted_iota(jnp.int32, d.shape, 1) + base
    arg = jnp.min(jnp.where(d == m, iota, jnp.int32(2 ** 30)), axis=1,
                  keepdims=True)
    return m, arg


def _dist_argmin_body(z_ref, emb2_ref, sz_ref, se_ref, idx_ref, loss_ref):
    i = pl.program_id(0)
    z16 = z_ref[...].astype(jnp.bfloat16)
    emb2 = emb2_ref[...]
    sz = sz_ref[...]
    se = se_ref[...]
    m1, a1 = _half_min_arg(sz, se[:, :_HALF], z16, emb2[:_HALF], 0)
    m2, a2 = _half_min_arg(sz, se[:, _HALF:], z16, emb2[_HALF:], _HALF)
    m1b = m1.astype(jnp.bfloat16).astype(jnp.float32)
    take2 = m2 < m1b
    idx_ref[...] = jnp.where(take2, a2, a1)[:, 0]
    chosen = jnp.where(take2, m2, m1)
    prev = jnp.where(i == 0, 0.0, loss_ref[0, 0])
    loss_ref[0, 0] = prev + jnp.sum(chosen)


def _tc_argmin(z_flat, embedding):
    n_tok = z_flat.shape[0]
    sz = jnp.sum(z_flat ** 2, axis=1, keepdims=True)
    se = jnp.sum(embedding ** 2, axis=1)[None, :]
    emb2 = embedding + embedding
    return pl.pallas_call(
        _dist_argmin_body,
        grid=(n_tok // _TM,),
        in_specs=[pl.BlockSpec((_TM, _DIM), lambda i: (i, 0)),
                  pl.BlockSpec((_NUM_CODE, _DIM), lambda i: (0, 0)),
                  pl.BlockSpec((_TM, 1), lambda i: (i, 0)),
                  pl.BlockSpec((1, _NUM_CODE), lambda i: (0, 0))],
        out_specs=[pl.BlockSpec((_TM,), lambda i: (i,)),
                   pl.BlockSpec((1, 1), lambda i: (0, 0),
                                memory_space=pltpu.SMEM)],
        out_shape=[jax.ShapeDtypeStruct((n_tok,), jnp.int32),
                   jax.ShapeDtypeStruct((1, 1), jnp.float32)],
        compiler_params=pltpu.CompilerParams(
            dimension_semantics=("arbitrary",)),
    )(z_flat, emb2, sz, se)


_GW = 128


def _sc_gather(emb_pad, idx):
    n_tok = idx.shape[0]
    info = plsc.get_sparse_core_info()
    nc, ns = info.num_cores, info.num_subcores
    nw = nc * ns
    bpw = n_tok // nw
    ch = 128
    mesh = plsc.VectorSubcoreMesh(core_axis_name="c", subcore_axis_name="s")

    @functools.partial(
        pl.kernel, mesh=mesh,
        out_type=jax.ShapeDtypeStruct((n_tok, _GW), jnp.float32),
        scratch_types=[pltpu.VMEM((bpw,), jnp.int32),
                       pltpu.VMEM((bpw, _GW), jnp.float32),
                       pltpu.SemaphoreType.DMA],
    )
    def k(emb_hbm, idx_hbm, out_hbm, idx_v, rows_v, sem):
        wid = lax.axis_index("s") * nc + lax.axis_index("c")
        base = wid * bpw
        pltpu.sync_copy(idx_hbm.at[pl.ds(base, bpw)], idx_v)
        for j in range(bpw // ch):
            pltpu.async_copy(emb_hbm.at[idx_v.at[pl.ds(j * ch, ch)]],
                             rows_v.at[pl.ds(j * ch, ch)], sem).wait()
        pltpu.sync_copy(rows_v, out_hbm.at[pl.ds(base, bpw)])

    return k(emb_pad, idx)


def kernel(z, embedding):
    b, n, d = z.shape
    z_flat = z.reshape(-1, d)
    idx, loss_sum = _tc_argmin(z_flat, embedding)
    emb_pad = jnp.pad(embedding, ((0, 0), (0, _GW - d)))
    z_q = _sc_gather(emb_pad, idx)[:, :d].reshape(b, n, d)
    loss = loss_sum[0, 0] * (2.0 / z_flat.size)
    z_q_st = z + (z_q - z)
    return (z_q_st, idx.reshape(b, n), loss)


# speedup vs baseline: 1.1209x; 1.1015x over previous
import functools

import jax
import jax.numpy as jnp
from jax import lax
from jax.experimental import pallas as pl
from jax.experimental.pallas import tpu as pltpu
from jax.experimental.pallas import tpu_sc as plsc

_NUM_CODE = 8192
_DIM = 32
_TM = 128


_HALF = _NUM_CODE // 2


def _half_min_arg(sz, se_h, z16, emb_h, base):
    dims = (((1,), (1,)), ((), ()))
    mm = lax.dot_general(z16, emb_h, dims,
                         preferred_element_type=jnp.float32)
    d = (sz + se_h) - 2.0 * mm
    m = jnp.min(d, axis=1, keepdims=True)
    iota = lax.broadcasted_iota(jnp.int32, d.shape, 1) + base
    arg = jnp.min(jnp.where(d == m, iota, jnp.int32(2 ** 30)), axis=1,
                  keepdims=True)
    return m, arg


def _dist_argmin_body(z_ref, emb_ref, sz_ref, se_ref, idx_ref, loss_ref):
    i = pl.program_id(0)
    z16 = z_ref[...].astype(jnp.bfloat16)
    emb = emb_ref[...]
    sz = sz_ref[...]
    se = se_ref[...]
    m1, a1 = _half_min_arg(sz, se[:, :_HALF], z16, emb[:_HALF], 0)
    m2, a2 = _half_min_arg(sz, se[:, _HALF:], z16, emb[_HALF:], _HALF)
    m1b = m1.astype(jnp.bfloat16).astype(jnp.float32)
    take2 = m2 < m1b
    idx_ref[...] = jnp.where(take2, a2, a1)[:, 0]
    chosen = jnp.where(take2, m2, m1)
    prev = jnp.where(i == 0, 0.0, loss_ref[0, 0])
    loss_ref[0, 0] = prev + jnp.sum(chosen)


def _tc_argmin(z_flat, embedding):
    n_tok = z_flat.shape[0]
    sz = jnp.sum(z_flat ** 2, axis=1, keepdims=True)
    se = jnp.sum(embedding ** 2, axis=1)[None, :]
    return pl.pallas_call(
        _dist_argmin_body,
        grid=(n_tok // _TM,),
        in_specs=[pl.BlockSpec((_TM, _DIM), lambda i: (i, 0)),
                  pl.BlockSpec((_NUM_CODE, _DIM), lambda i: (0, 0)),
                  pl.BlockSpec((_TM, 1), lambda i: (i, 0)),
                  pl.BlockSpec((1, _NUM_CODE), lambda i: (0, 0))],
        out_specs=[pl.BlockSpec((_TM,), lambda i: (i,)),
                   pl.BlockSpec((1, 1), lambda i: (0, 0),
                                memory_space=pltpu.SMEM)],
        out_shape=[jax.ShapeDtypeStruct((n_tok,), jnp.int32),
                   jax.ShapeDtypeStruct((1, 1), jnp.float32)],
        compiler_params=pltpu.CompilerParams(
            dimension_semantics=("arbitrary",)),
    )(z_flat, embedding, sz, se)


_GW = 128


def _sc_gather(emb_pad, idx):
    n_tok = idx.shape[0]
    info = plsc.get_sparse_core_info()
    nc, ns = info.num_cores, info.num_subcores
    nw = nc * ns
    bpw = n_tok // nw
    ch = 128
    mesh = plsc.VectorSubcoreMesh(core_axis_name="c", subcore_axis_name="s")

    @functools.partial(
        pl.kernel, mesh=mesh,
        out_type=jax.ShapeDtypeStruct((n_tok, _GW), jnp.float32),
        scratch_types=[pltpu.VMEM((bpw,), jnp.int32),
                       pltpu.VMEM((bpw, _GW), jnp.float32),
                       pltpu.SemaphoreType.DMA],
    )
    def k(emb_hbm, idx_hbm, out_hbm, idx_v, rows_v, sem):
        wid = lax.axis_index("s") * nc + lax.axis_index("c")
        base = wid * bpw
        pltpu.sync_copy(idx_hbm.at[pl.ds(base, bpw)], idx_v)
        for j in range(bpw // ch):
            pltpu.async_copy(emb_hbm.at[idx_v.at[pl.ds(j * ch, ch)]],
                             rows_v.at[pl.ds(j * ch, ch)], sem).wait()
        pltpu.sync_copy(rows_v, out_hbm.at[pl.ds(base, bpw)])

    return k(emb_pad, idx)


def kernel(z, embedding):
    b, n, d = z.shape
    z_flat = z.reshape(-1, d)
    idx, loss_sum = _tc_argmin(z_flat, embedding)
    emb_pad = jnp.pad(embedding, ((0, 0), (0, _GW - d)))
    z_q = _sc_gather(emb_pad, idx)[:, :d].reshape(b, n, d)
    loss = loss_sum[0, 0] * (2.0 / z_flat.size)
    z_q_st = z + (z_q - z)
    return (z_q_st, idx.reshape(b, n), loss)
